# baseline (speedup 1.0000x reference)
import functools

import jax
import jax.numpy as jnp
from jax import lax
from jax.experimental import pallas as pl
from jax.experimental.pallas import tpu as pltpu
from jax.experimental.pallas import tpu_sc as plsc

_VOCAB = 1_000_000
_D = 64
_B = 16384
_W = 128
_NC = 2
_NS = 16
_L = 16
_NW = _NC * _NS
_BPW = _B // _NW
_NG = _BPW // _L
_R = 6

_mesh = plsc.VectorSubcoreMesh(
    core_axis_name="c", subcore_axis_name="s", num_cores=_NC, num_subcores=_NS
)


@functools.partial(
    pl.kernel,
    out_type=jax.ShapeDtypeStruct((_B,), jnp.float32),
    mesh=_mesh,
    compiler_params=pltpu.CompilerParams(needs_layout_passes=False),
    scratch_types=[
        pltpu.VMEM((_BPW,), jnp.int32),
        pltpu.VMEM((_BPW,), jnp.int32),
        pltpu.VMEM((_R, _D, _W), jnp.float32),
        pltpu.VMEM((_R, _D, _W), jnp.float32),
        pltpu.VMEM((_L, _W), jnp.float32),
        pltpu.VMEM((_L, _W), jnp.float32),
        pltpu.VMEM((_L, _D), jnp.float32),
        pltpu.VMEM((_L, _D), jnp.float32),
        pltpu.VMEM((_BPW,), jnp.float32),
        [pltpu.SemaphoreType.DMA] * _R,
        [pltpu.SemaphoreType.DMA] * _R,
        pltpu.SemaphoreType.DMA,
        pltpu.SemaphoreType.DMA,
    ],
)
def _glove_sc(left_hbm, right_hbm, ltab_hbm, lbias_hbm, rtab_hbm, rbias_hbm,
              out_hbm, lids, rids, lblk, rblk, lbst, rbst, lstage, rstage,
              outv, lsems, rsems, lbsem, rbsem):
    wid = lax.axis_index("s") * _NC + lax.axis_index("c")
    base = wid * _BPW

    pltpu.sync_copy(left_hbm.at[pl.ds(base, _BPW)], lids)
    pltpu.sync_copy(right_hbm.at[pl.ds(base, _BPW)], rids)

    lane = lax.iota(jnp.int32, _L)
    cvec = lax.iota(jnp.int32, _L)

    def group(g, carry):
        gb = pl.multiple_of(g * _L, _L)
        vl = lids[pl.ds(gb, _L)]
        vr = rids[pl.ds(gb, _L)]
        lblks = (vl >> 7) << 7
        rblks = (vr >> 7) << 7
        llanes = vl & (_W - 1)
        rlanes = vr & (_W - 1)

        def fire(j):
            slot = j % _R
            cl = pltpu.async_copy(
                ltab_hbm.at[:, pl.ds(pl.multiple_of(lblks[j], _W), _W)],
                lblk.at[slot], lsems[slot])
            cr = pltpu.async_copy(
                rtab_hbm.at[:, pl.ds(pl.multiple_of(rblks[j], _W), _W)],
                rblk.at[slot], rsems[slot])
            return cl, cr

        inflight = [fire(j) for j in range(_R)]

        bias_copies = []
        for j in range(_L):
            bias_copies.append(pltpu.async_copy(
                lbias_hbm.at[:, pl.ds(pl.multiple_of(lblks[j], _W), _W)],
                lbst.at[pl.ds(j, 1)], lbsem))
            bias_copies.append(pltpu.async_copy(
                rbias_hbm.at[:, pl.ds(pl.multiple_of(rblks[j], _W), _W)],
                rbst.at[pl.ds(j, 1)], rbsem))
        for j in range(_L):
            slot = j % _R
            cl, cr = inflight[j]
            cl.wait()
            cr.wait()
            lj = llanes[j]
            rj = rlanes[j]
            for cc in range(0, _D, _L):
                lstage[j, pl.ds(cc, _L)] = plsc.load_gather(
                    lblk.at[slot], [cvec + cc, jnp.full((_L,), 0, jnp.int32) + lj])
                rstage[j, pl.ds(cc, _L)] = plsc.load_gather(
                    rblk.at[slot], [cvec + cc, jnp.full((_L,), 0, jnp.int32) + rj])
            if j + _R < _L:
                inflight.append(fire(j + _R))

        for c in bias_copies:
            c.wait()

        acc = plsc.load_gather(lbst, [lane, llanes]) + plsc.load_gather(
            rbst, [lane, rlanes])
        for c in range(_D):
            col = jnp.full((_L,), c, jnp.int32)
            acc = acc + plsc.load_gather(lstage, [lane, col]) * plsc.load_gather(
                rstage, [lane, col])
        outv[pl.ds(gb, _L)] = acc
        return carry

    lax.fori_loop(0, _NG, group, 0)

    pltpu.sync_copy(outv, out_hbm.at[pl.ds(base, _BPW)])


def kernel(left_id, right_id, l_emb, l_bias, r_emb, r_bias):
    return _glove_sc(
        left_id.astype(jnp.int32), right_id.astype(jnp.int32),
        l_emb.T, l_bias.T, r_emb.T, r_bias.T,
    )


# speedup vs baseline: 1.0298x; 1.0298x over previous
import functools

import jax
import jax.numpy as jnp
from jax import lax
from jax.experimental import pallas as pl
from jax.experimental.pallas import tpu as pltpu
from jax.experimental.pallas import tpu_sc as plsc

_VOCAB = 1_000_000
_D = 64
_B = 16384
_W = 128
_NC = 2
_NS = 16
_L = 16
_NW = _NC * _NS
_BPW = _B // _NW
_NG = _BPW // _L
_R = 6

_mesh = plsc.VectorSubcoreMesh(
    core_axis_name="c", subcore_axis_name="s", num_cores=_NC, num_subcores=_NS
)


@functools.partial(
    pl.kernel,
    out_type=jax.ShapeDtypeStruct((_B,), jnp.float32),
    mesh=_mesh,
    compiler_params=pltpu.CompilerParams(needs_layout_passes=False),
    scratch_types=[
        pltpu.VMEM((_BPW,), jnp.int32),
        pltpu.VMEM((_BPW,), jnp.int32),
        pltpu.VMEM((_R, _D, _W), jnp.float32),
        pltpu.VMEM((_R, _D, _W), jnp.float32),
        pltpu.VMEM((_L, _W), jnp.float32),
        pltpu.VMEM((_L, _W), jnp.float32),
        pltpu.VMEM((_L, _D), jnp.float32),
        pltpu.VMEM((_L, _D), jnp.float32),
        pltpu.VMEM((_BPW,), jnp.float32),
        [pltpu.SemaphoreType.DMA] * _R,
        [pltpu.SemaphoreType.DMA] * _R,
        pltpu.SemaphoreType.DMA,
        pltpu.SemaphoreType.DMA,
    ],
)
def _glove_sc(left_hbm, right_hbm, ltab_hbm, lbias_hbm, rtab_hbm, rbias_hbm,
              out_hbm, lids, rids, lblk, rblk, lbst, rbst, lstage, rstage,
              outv, lsems, rsems, lbsem, rbsem):
    wid = lax.axis_index("s") * _NC + lax.axis_index("c")
    base = wid * _BPW

    pltpu.sync_copy(left_hbm.at[pl.ds(base, _BPW)], lids)
    pltpu.sync_copy(right_hbm.at[pl.ds(base, _BPW)], rids)

    lane = lax.iota(jnp.int32, _L)
    cvec = lax.iota(jnp.int32, _L)

    def group(g, carry):
        gb = pl.multiple_of(g * _L, _L)
        vl = lids[pl.ds(gb, _L)]
        vr = rids[pl.ds(gb, _L)]
        lblks = (vl >> 7) << 7
        rblks = (vr >> 7) << 7
        llanes = vl & (_W - 1)
        rlanes = vr & (_W - 1)

        def fire(j):
            slot = j % _R
            cl = pltpu.async_copy(
                ltab_hbm.at[:, pl.ds(pl.multiple_of(lblks[j], _W), _W)],
                lblk.at[slot], lsems[slot])
            cr = pltpu.async_copy(
                rtab_hbm.at[:, pl.ds(pl.multiple_of(rblks[j], _W), _W)],
                rblk.at[slot], rsems[slot])
            return cl, cr

        bias_copies = []
        for j in range(_L):
            bias_copies.append(pltpu.async_copy(
                lbias_hbm.at[:, pl.ds(pl.multiple_of(lblks[j], _W), _W)],
                lbst.at[pl.ds(j, 1)], lbsem))
            bias_copies.append(pltpu.async_copy(
                rbias_hbm.at[:, pl.ds(pl.multiple_of(rblks[j], _W), _W)],
                rbst.at[pl.ds(j, 1)], rbsem))

        inflight = [fire(j) for j in range(_R)]
        for j in range(_L):
            slot = j % _R
            cl, cr = inflight[j]
            cl.wait()
            cr.wait()
            lj = llanes[j]
            rj = rlanes[j]
            for cc in range(0, _D, _L):
                lstage[j, pl.ds(cc, _L)] = plsc.load_gather(
                    lblk.at[slot], [cvec + cc, jnp.full((_L,), 0, jnp.int32) + lj])
                rstage[j, pl.ds(cc, _L)] = plsc.load_gather(
                    rblk.at[slot], [cvec + cc, jnp.full((_L,), 0, jnp.int32) + rj])
            if j + _R < _L:
                inflight.append(fire(j + _R))

        for c in bias_copies:
            c.wait()

        acc = plsc.load_gather(lbst, [lane, llanes]) + plsc.load_gather(
            rbst, [lane, rlanes])
        for c in range(_D):
            col = jnp.full((_L,), c, jnp.int32)
            acc = acc + plsc.load_gather(lstage, [lane, col]) * plsc.load_gather(
                rstage, [lane, col])
        outv[pl.ds(gb, _L)] = acc
        return carry

    lax.fori_loop(0, _NG, group, 0)

    pltpu.sync_copy(outv, out_hbm.at[pl.ds(base, _BPW)])


def kernel(left_id, right_id, l_emb, l_bias, r_emb, r_bias):
    return _glove_sc(
        left_id.astype(jnp.int32), right_id.astype(jnp.int32),
        l_emb.T, l_bias.T, r_emb.T, r_bias.T,
    )
